# Initial kernel scaffold; baseline (speedup 1.0000x reference)
import functools

import jax
import jax.numpy as jnp
from jax import lax
from jax.experimental import pallas as pl
from jax.experimental.pallas import tpu as pltpu
from jax.experimental.pallas import tpu_sc as plsc

_BATCH = 16384
_FIELD = 26
_K = 16
_NC = 2
_NS = 16
_NW = _NC * _NS
_ROWS_W = _BATCH // _NW
_CH = 64
_NCH = _ROWS_W // _CH
_ICH = _CH * _FIELD
_GW = 128
_G = _ICH // _GW


def _fm_body(idx_hbm, fv_hbm, lw_hbm, bias_hbm, fm_hbm, out_hbm,
             idx_v, fv_v, rows_v, lin_v, z_v, bias_v, gsem, lsem):
    wid = lax.axis_index("s") * _NC + lax.axis_index("c")
    base = wid * _ROWS_W

    pltpu.sync_copy(bias_hbm, bias_v)

    def chunk_body(c, carry):
        rbase = base + c * _CH
        ibase = rbase * _FIELD

        pltpu.sync_copy(idx_hbm.at[pl.ds(ibase // _GW, _G)], idx_v)
        pltpu.sync_copy(fv_hbm.at[pl.ds(ibase, _ICH)], fv_v)

        gcopies = []
        lcopies = []
        for g in range(_G):
            gcopies.append(pltpu.async_copy(
                fm_hbm.at[idx_v.at[g]], rows_v.at[pl.ds(g * _GW, _GW)],
                gsem))
            lcopies.append(pltpu.async_copy(
                lw_hbm.at[idx_v.at[g]], lin_v.at[pl.ds(g * _GW, _GW)],
                lsem))
        for cp in gcopies:
            cp.wait()
        for cp in lcopies:
            cp.wait()

        def row_body(r, carry2):
            ib = r * _FIELD
            zero = jnp.zeros((_K,), jnp.float32)
            acc0, acc1 = zero, zero
            sq0, sq1 = zero, zero
            lin = jnp.float32(0.0)
            for f in range(_FIELD):
                row = rows_v[ib + f, :]
                v = fv_v[ib + f]
                xv = row * v
                if f % 2 == 0:
                    acc0 = acc0 + xv
                    sq0 = sq0 + xv * xv
                else:
                    acc1 = acc1 + xv
                    sq1 = sq1 + xv * xv
                lin = lin + lin_v[ib + f, 0] * v
            s = acc0 + acc1
            q = sq0 + sq1
            second = 0.5 * jnp.sum(s * s - q)
            z_v[r] = lin + second
            return carry2

        lax.fori_loop(0, _CH, row_body, 0)

        bias_vec = bias_v[:]
        for i in range(_CH // _K):
            zz = z_v[pl.ds(i * _K, _K)] + bias_vec
            z_v[pl.ds(i * _K, _K)] = 1.0 / (1.0 + jnp.exp(-zz))
        pltpu.sync_copy(z_v, out_hbm.at[pl.ds(rbase, _CH)])
        return carry

    lax.fori_loop(0, _NCH, chunk_body, 0)


@jax.jit
def _fm_sc(idx2d, fv_flat, lw, bias16, fm):
    mesh = plsc.VectorSubcoreMesh(core_axis_name="c", subcore_axis_name="s",
                                  num_cores=_NC, num_subcores=_NS)
    f = pl.kernel(
        _fm_body,
        out_type=jax.ShapeDtypeStruct((_BATCH,), jnp.float32),
        mesh=mesh,
        scratch_types=[
            pltpu.VMEM((_G, _GW), jnp.int32),
            pltpu.VMEM((_ICH,), jnp.float32),
            pltpu.VMEM((_ICH, _K), jnp.float32),
            pltpu.VMEM((_ICH, 1), jnp.float32),
            pltpu.VMEM((_CH,), jnp.float32),
            pltpu.VMEM((_K,), jnp.float32),
            pltpu.SemaphoreType.DMA,
            pltpu.SemaphoreType.DMA,
        ],
    )
    return f(idx2d, fv_flat, lw, bias16, fm)


def kernel(feat_index, feat_value, linear_weight, linear_bias, fm_weight):
    idx2d = feat_index.reshape(_BATCH * _FIELD // _GW, _GW)
    fv_flat = feat_value.reshape(_BATCH * _FIELD)
    bias16 = jnp.broadcast_to(linear_bias.reshape(()), (_K,))
    out = _fm_sc(idx2d, fv_flat, linear_weight, bias16, fm_weight)
    return out.reshape(_BATCH, 1)


# speedup vs baseline: 1.2935x; 1.2935x over previous
import jax
import jax.numpy as jnp
from jax import lax
from jax.experimental import pallas as pl
from jax.experimental.pallas import tpu as pltpu
from jax.experimental.pallas import tpu_sc as plsc

_BATCH = 16384
_FIELD = 26
_K = 16
_NC = 2
_NS = 16
_NW = _NC * _NS
_ROWS_W = _BATCH // _NW
_CH = 64
_NCH = _ROWS_W // _CH
_ICH = _CH * _FIELD
_IPAD = _ICH + 16
_GW = 128
_G = _ICH // _GW


def _fm_body(idx_hbm, fv_hbm, lw_hbm, bias_hbm, fm_hbm, out_hbm,
             idx_v, fv_v, rows_v, lin_v, z_v, bias_v, gsem, lsem):
    wid = lax.axis_index("s") * _NC + lax.axis_index("c")
    base = wid * _ROWS_W

    pltpu.sync_copy(bias_hbm, bias_v)
    iota = lax.iota(jnp.int32, _K)
    tail_mask = iota < (_FIELD - _K)

    def chunk_body(c, carry):
        rbase = base + c * _CH
        ibase = rbase * _FIELD

        pltpu.sync_copy(idx_hbm.at[pl.ds(ibase, _ICH)], idx_v)
        pltpu.sync_copy(fv_hbm.at[pl.ds(ibase, _ICH)],
                        fv_v.at[pl.ds(0, _ICH)])

        gcopies = []
        lcopies = []
        for g in range(_G):
            gcopies.append(pltpu.async_copy(
                fm_hbm.at[idx_v.at[pl.ds(g * _GW, _GW)]],
                rows_v.at[pl.ds(g * _GW, _GW)], gsem))
            lcopies.append(pltpu.async_copy(
                lw_hbm.at[idx_v.at[pl.ds(g * _GW, _GW)]],
                lin_v.at[pl.ds(g * _GW, _GW)], lsem))
        for cp in gcopies:
            cp.wait()
        for cp in lcopies:
            cp.wait()

        def row_body(r, z_vec):
            ib = r * _FIELD
            fv0 = fv_v[pl.ds(ib, _K)]
            fv1 = fv_v[pl.ds(ib + _K, _K)]
            l0 = lin_v[pl.ds(ib, _K)]
            l1 = lin_v[pl.ds(ib + _K, _K)]
            zero = jnp.zeros((_K,), jnp.float32)
            acc = [zero, zero]
            sq = [zero, zero]
            for f in range(_FIELD):
                v = fv0[f] if f < _K else fv1[f - _K]
                xv = rows_v[ib + f, :] * v
                acc[f % 2] = acc[f % 2] + xv
                sq[f % 2] = sq[f % 2] + xv * xv
            s = acc[0] + acc[1]
            q = sq[0] + sq[1]
            lin_part = l0 * fv0 + jnp.where(tail_mask, l1 * fv1, 0.0)
            z = jnp.sum(0.5 * (s * s - q) + lin_part)
            z_vec = jnp.where(iota == (r & (_K - 1)), z, z_vec)

            @pl.when((r & (_K - 1)) == (_K - 1))
            def _():
                zz = z_vec + bias_v[:]
                z_v[pl.ds(r - (_K - 1), _K)] = 1.0 / (1.0 + jnp.exp(-zz))

            return z_vec

        lax.fori_loop(0, _CH, row_body, jnp.zeros((_K,), jnp.float32))
        pltpu.sync_copy(z_v, out_hbm.at[pl.ds(rbase, _CH)])
        return carry

    lax.fori_loop(0, _NCH, chunk_body, 0)


@jax.jit
def _fm_sc(idx2d, fv_flat, lw_flat, bias16, fm):
    mesh = plsc.VectorSubcoreMesh(core_axis_name="c", subcore_axis_name="s",
                                  num_cores=_NC, num_subcores=_NS)
    f = pl.kernel(
        _fm_body,
        out_type=jax.ShapeDtypeStruct((_BATCH,), jnp.float32),
        mesh=mesh,
        compiler_params=pltpu.CompilerParams(needs_layout_passes=False,
                                             use_tc_tiling_on_sc=False),
        scratch_types=[
            pltpu.VMEM((_ICH,), jnp.int32),
            pltpu.VMEM((_IPAD,), jnp.float32),
            pltpu.VMEM((_ICH, _K), jnp.float32),
            pltpu.VMEM((_IPAD,), jnp.float32),
            pltpu.VMEM((_CH,), jnp.float32),
            pltpu.VMEM((_K,), jnp.float32),
            pltpu.SemaphoreType.DMA,
            pltpu.SemaphoreType.DMA,
        ],
    )
    return f(idx2d, fv_flat, lw_flat, bias16, fm)


def kernel(feat_index, feat_value, linear_weight, linear_bias, fm_weight):
    idx2d = feat_index.reshape(_BATCH * _FIELD)
    fv_flat = feat_value.reshape(_BATCH * _FIELD)
    lw_flat = linear_weight.reshape(-1)
    bias16 = jnp.broadcast_to(linear_bias.reshape(()), (_K,))
    out = _fm_sc(idx2d, fv_flat, lw_flat, bias16, fm_weight)
    return out.reshape(_BATCH, 1)
